# Initial kernel scaffold; baseline (speedup 1.0000x reference)
import jax
import jax.numpy as jnp
from jax.experimental import pallas as pl


def kernel(edge_index, edge_attr, simp_edge_batch):
    raise NotImplementedError("write your pallas kernel here")


# speedup vs baseline: 7.3865x; 7.3865x over previous
import functools

import jax
import jax.numpy as jnp
from jax import lax
from jax.experimental import pallas as pl
from jax.experimental.pallas import tpu as pltpu
from jax.experimental.pallas import tpu_sc as plsc

S_OUT = 800_000
D = 17
WS = 2000
NWIN = S_OUT // WS
NTILES = 32
C = 1024
FC = 1008
ACC_N = WS * 20 + 32
BOUNDS_PAD = NWIN + 32


def _build(E):
    mesh = plsc.VectorSubcoreMesh(core_axis_name="c", subcore_axis_name="s")

    @functools.partial(
        pl.kernel,
        out_type=(
            jax.ShapeDtypeStruct((S_OUT,), jnp.int32),
            jax.ShapeDtypeStruct((S_OUT,), jnp.int32),
            jax.ShapeDtypeStruct((S_OUT, D), jnp.float32),
            jax.ShapeDtypeStruct((E,), jnp.int32),
        ),
        mesh=mesh,
        compiler_params=pltpu.CompilerParams(
            needs_layout_passes=False, use_tc_tiling_on_sc=False),
        scratch_types=[
            pltpu.VMEM((BOUNDS_PAD,), jnp.int32),
            pltpu.VMEM((C,), jnp.int32),
            pltpu.VMEM((C,), jnp.int32),
            pltpu.VMEM((C,), jnp.int32),
            pltpu.VMEM((C, D), jnp.float32),
            pltpu.VMEM((ACC_N,), jnp.float32),
            pltpu.VMEM((FC, D), jnp.float32),
            pltpu.VMEM((FC,), jnp.int32),
            pltpu.VMEM((FC,), jnp.int32),
        ],
    )
    def k(ei0_hbm, ei1_hbm, attr_hbm, seg_hbm, bounds_hbm,
          out_e0, out_e1, out_attr, out_inv,
          boundsv, segbuf, ei0buf, ei1buf, attrbuf, acc, fat, fei0, fei1):
        cid = lax.axis_index("c")
        sid = lax.axis_index("s")
        wid = sid * 2 + cid
        lane = lax.iota(jnp.int32, 16)
        ones = jnp.ones((16,), jnp.float32)

        pltpu.sync_copy(bounds_hbm, boundsv)

        base_win = NWIN // NTILES
        extra = NWIN - base_win * NTILES
        nwin_t = jnp.where(wid < extra, base_win + 1, base_win)

        def window_body(kwin, _):
            win = wid + kwin * NTILES
            wbase = win * WS
            bv = boundsv[pl.ds(win, 16)]
            ew0 = bv[0]
            ew1 = bv[1]

            def zero_body(i, _):
                acc[pl.ds(i * 16, 16)] = jnp.zeros((16,), jnp.float32)
                return 0

            lax.fori_loop(0, ACC_N // 16, zero_body, 0)

            ea0 = ew0 & ~7
            nblk = (ew1 - ea0 + (C - 1)) >> 10

            def edge_block(b, _):
                e0 = ea0 + b * C
                e0c = pl.multiple_of(jnp.minimum(e0, E - C), 8)
                pltpu.sync_copy(seg_hbm.at[pl.ds(e0c, C)], segbuf)
                pltpu.sync_copy(attr_hbm.at[pl.ds(e0c, C), :], attrbuf)
                pltpu.sync_copy(ei0_hbm.at[pl.ds(e0c, C)], ei0buf)
                pltpu.sync_copy(ei1_hbm.at[pl.ds(e0c, C)], ei1buf)
                pltpu.sync_copy(segbuf, out_inv.at[pl.ds(e0c, C)])

                def group(g, _):
                    base = g * 16
                    rows = base + lane
                    seg16 = segbuf[pl.ds(base, 16)]
                    ge = e0c + rows
                    valid = (ge >= jnp.maximum(ew0, e0)) & (ge < ew1)
                    idxf = (seg16 - wbase) * 20
                    for j in range(D):
                        v = plsc.load_gather(
                            attrbuf, [rows, jnp.full((16,), j, jnp.int32)])
                        plsc.addupdate_scatter(acc, [idxf + j], v, mask=valid)
                    v0 = ei0buf[pl.ds(base, 16)].astype(jnp.float32)
                    plsc.addupdate_scatter(acc, [idxf + D], v0, mask=valid)
                    v1 = ei1buf[pl.ds(base, 16)].astype(jnp.float32)
                    plsc.addupdate_scatter(acc, [idxf + (D + 1)], v1,
                                           mask=valid)
                    plsc.addupdate_scatter(acc, [idxf + (D + 2)], ones,
                                           mask=valid)
                    return 0

                lax.fori_loop(0, C // 16, group, 0)
                return 0

            lax.fori_loop(0, nblk, edge_block, 0)

            for l0 in (0, WS - FC):
                g0 = wbase + l0

                def fgroup(g, _):
                    r = (l0 + g * 16) + lane
                    frows = g * 16 + lane
                    a = r * 20
                    cnt = plsc.load_gather(acc, [a + (D + 2)])
                    rcp = 1.0 / cnt
                    ts = plsc.load_gather(acc, [a])
                    plsc.store_scatter(
                        fat, [frows, jnp.zeros((16,), jnp.int32)], ts * rcp)
                    for j in range(1, D):
                        v = plsc.load_gather(acc, [a + j])
                        plsc.store_scatter(
                            fat, [frows, jnp.full((16,), j, jnp.int32)], v)
                    e0v = plsc.load_gather(acc, [a + D]) * rcp
                    fei0[pl.ds(g * 16, 16)] = e0v.astype(jnp.int32)
                    e1v = plsc.load_gather(acc, [a + (D + 1)]) * rcp
                    fei1[pl.ds(g * 16, 16)] = e1v.astype(jnp.int32)
                    return 0

                lax.fori_loop(0, FC // 16, fgroup, 0)
                pltpu.sync_copy(fat, out_attr.at[pl.ds(g0, FC), :])
                pltpu.sync_copy(fei0, out_e0.at[pl.ds(g0, FC)])
                pltpu.sync_copy(fei1, out_e1.at[pl.ds(g0, FC)])
            return 0

        lax.fori_loop(0, nwin_t, window_body, 0)

    return k


@jax.jit
def kernel(edge_index, edge_attr, simp_edge_batch):
    E = edge_attr.shape[0]
    starts = jnp.arange(NWIN + 1, dtype=jnp.int32) * WS
    bounds = jnp.searchsorted(simp_edge_batch, starts, side="left")
    bounds = bounds.astype(jnp.int32)
    bounds = jnp.concatenate(
        [bounds, jnp.zeros((BOUNDS_PAD - NWIN - 1,), jnp.int32)])
    out_e0, out_e1, out_attr, out_inv = _build(E)(
        edge_index[0], edge_index[1], edge_attr, simp_edge_batch, bounds)
    return jnp.stack([out_e0, out_e1]), out_attr, out_inv


# speedup vs baseline: 7.9172x; 1.0718x over previous
import functools

import jax
import jax.numpy as jnp
from jax import lax
from jax.experimental import pallas as pl
from jax.experimental.pallas import tpu as pltpu
from jax.experimental.pallas import tpu_sc as plsc

S_OUT = 800_000
D = 17
WS = 2000
NWIN = S_OUT // WS
NTILES = 32
C = 1024
FC = 1008
ACC_N = WS * 20 + 32
BOUNDS_PAD = NWIN + 32


def _build(E):
    mesh = plsc.VectorSubcoreMesh(core_axis_name="c", subcore_axis_name="s")

    @functools.partial(
        pl.kernel,
        out_type=(
            jax.ShapeDtypeStruct((S_OUT,), jnp.int32),
            jax.ShapeDtypeStruct((S_OUT,), jnp.int32),
            jax.ShapeDtypeStruct((S_OUT, D), jnp.float32),
            jax.ShapeDtypeStruct((E,), jnp.int32),
        ),
        mesh=mesh,
        compiler_params=pltpu.CompilerParams(
            needs_layout_passes=False, use_tc_tiling_on_sc=False),
        scratch_types=[
            pltpu.VMEM((BOUNDS_PAD,), jnp.int32),
            pltpu.VMEM((2 * C,), jnp.int32),
            pltpu.VMEM((2 * C,), jnp.int32),
            pltpu.VMEM((2 * C,), jnp.int32),
            pltpu.VMEM((2 * C, D), jnp.float32),
            pltpu.VMEM((ACC_N,), jnp.float32),
            pltpu.VMEM((FC, D), jnp.float32),
            pltpu.VMEM((FC,), jnp.int32),
            pltpu.VMEM((FC,), jnp.int32),
            pltpu.SemaphoreType.DMA,
            pltpu.SemaphoreType.DMA,
            pltpu.SemaphoreType.DMA,
            pltpu.SemaphoreType.DMA,
        ],
    )
    def k(ei0_hbm, ei1_hbm, attr_hbm, seg_hbm, bounds_hbm,
          out_e0, out_e1, out_attr, out_inv,
          boundsv, segbuf, ei0buf, ei1buf, attrbuf, acc, fat, fei0, fei1,
          insem0, insem1, invsem0, invsem1):
        cid = lax.axis_index("c")
        sid = lax.axis_index("s")
        wid = sid * 2 + cid
        lane = lax.iota(jnp.int32, 16)
        ones = jnp.ones((16,), jnp.float32)

        pltpu.sync_copy(bounds_hbm, boundsv)

        base_win = NWIN // NTILES
        extra = NWIN - base_win * NTILES
        nwin_t = jnp.where(wid < extra, base_win + 1, base_win)

        def window_body(kwin, _):
            win = wid + kwin * NTILES
            wbase = win * WS
            bv = boundsv[pl.ds(win, 16)]
            ew0 = bv[0]
            ew1 = bv[1]

            ea0 = ew0 & ~7
            nblk = (ew1 - ea0 + (C - 1)) >> 10

            def eoff(b):
                return pl.multiple_of(
                    jnp.minimum(ea0 + b * C, E - C), 8)

            insems = (insem0, insem1)
            invsems = (invsem0, invsem1)

            def issue_in(b, slot):
                e0c = eoff(b)
                so = slot * C
                sem = insems[slot]
                pltpu.async_copy(seg_hbm.at[pl.ds(e0c, C)],
                                 segbuf.at[pl.ds(so, C)], sem)
                pltpu.async_copy(attr_hbm.at[pl.ds(e0c, C), :],
                                 attrbuf.at[pl.ds(so, C), :], sem)
                pltpu.async_copy(ei0_hbm.at[pl.ds(e0c, C)],
                                 ei0buf.at[pl.ds(so, C)], sem)
                pltpu.async_copy(ei1_hbm.at[pl.ds(e0c, C)],
                                 ei1buf.at[pl.ds(so, C)], sem)

            def wait_in(slot):
                sem = insems[slot]
                pltpu.make_async_copy(
                    seg_hbm.at[pl.ds(0, C)], segbuf.at[pl.ds(0, C)],
                    sem).wait()
                pltpu.make_async_copy(
                    attr_hbm.at[pl.ds(0, C), :], attrbuf.at[pl.ds(0, C), :],
                    sem).wait()
                pltpu.make_async_copy(
                    ei0_hbm.at[pl.ds(0, C)], ei0buf.at[pl.ds(0, C)],
                    sem).wait()
                pltpu.make_async_copy(
                    ei1_hbm.at[pl.ds(0, C)], ei1buf.at[pl.ds(0, C)],
                    sem).wait()

            def issue_inv(b, slot):
                pltpu.async_copy(segbuf.at[pl.ds(slot * C, C)],
                                 out_inv.at[pl.ds(eoff(b), C)],
                                 invsems[slot])

            def wait_inv(slot):
                pltpu.make_async_copy(
                    segbuf.at[pl.ds(0, C)], out_inv.at[pl.ds(0, C)],
                    invsems[slot]).wait()

            def compute(b, slot):
                e0 = ea0 + b * C
                e0c = eoff(b)
                so = slot * C
                lo = jnp.maximum(ew0, e0)

                def group(g, _):
                    base = so + g * 16
                    rows = base + lane
                    seg16 = segbuf[pl.ds(base, 16)]
                    ge = (e0c - so) + rows
                    valid = (ge >= lo) & (ge < ew1)
                    idxf = (seg16 - wbase) * 20
                    for j in range(D):
                        v = plsc.load_gather(
                            attrbuf, [rows, jnp.full((16,), j, jnp.int32)])
                        plsc.addupdate_scatter(acc, [idxf + j], v, mask=valid)
                    v0 = ei0buf[pl.ds(base, 16)].astype(jnp.float32)
                    plsc.addupdate_scatter(acc, [idxf + D], v0, mask=valid)
                    v1 = ei1buf[pl.ds(base, 16)].astype(jnp.float32)
                    plsc.addupdate_scatter(acc, [idxf + (D + 1)], v1,
                                           mask=valid)
                    plsc.addupdate_scatter(acc, [idxf + (D + 2)], ones,
                                           mask=valid)
                    return 0

                lax.fori_loop(0, C // 16, group, 0)

            issue_in(0, 0)

            def zero_body(i, _):
                acc[pl.ds(i * 16, 16)] = jnp.zeros((16,), jnp.float32)
                return 0

            lax.fori_loop(0, ACC_N // 16, zero_body, 0)
            npair = (nblk + 1) >> 1

            def pair_body(p, _):
                b0 = 2 * p
                b1 = b0 + 1
                wait_in(0)
                issue_inv(b0, 0)

                @pl.when(b1 < nblk)
                def _():
                    @pl.when(b0 >= 1)
                    def _():
                        wait_inv(1)
                    issue_in(b1, 1)

                compute(b0, 0)

                @pl.when(b1 < nblk)
                def _():
                    wait_in(1)
                    issue_inv(b1, 1)

                    @pl.when(b1 + 1 < nblk)
                    def _():
                        wait_inv(0)
                        issue_in(b1 + 1, 0)

                    compute(b1, 1)
                return 0

            lax.fori_loop(0, npair, pair_body, 0)
            wait_inv(0)
            wait_inv(1)

            for l0 in (0, WS - FC):
                g0 = wbase + l0

                def fgroup(g, _):
                    r = (l0 + g * 16) + lane
                    frows = g * 16 + lane
                    a = r * 20
                    cnt = plsc.load_gather(acc, [a + (D + 2)])
                    rcp = 1.0 / cnt
                    ts = plsc.load_gather(acc, [a])
                    plsc.store_scatter(
                        fat, [frows, jnp.zeros((16,), jnp.int32)], ts * rcp)
                    for j in range(1, D):
                        v = plsc.load_gather(acc, [a + j])
                        plsc.store_scatter(
                            fat, [frows, jnp.full((16,), j, jnp.int32)], v)
                    e0v = plsc.load_gather(acc, [a + D]) * rcp
                    fei0[pl.ds(g * 16, 16)] = e0v.astype(jnp.int32)
                    e1v = plsc.load_gather(acc, [a + (D + 1)]) * rcp
                    fei1[pl.ds(g * 16, 16)] = e1v.astype(jnp.int32)
                    return 0

                lax.fori_loop(0, FC // 16, fgroup, 0)
                pltpu.sync_copy(fat, out_attr.at[pl.ds(g0, FC), :])
                pltpu.sync_copy(fei0, out_e0.at[pl.ds(g0, FC)])
                pltpu.sync_copy(fei1, out_e1.at[pl.ds(g0, FC)])
            return 0

        lax.fori_loop(0, nwin_t, window_body, 0)

    return k


@jax.jit
def kernel(edge_index, edge_attr, simp_edge_batch):
    E = edge_attr.shape[0]
    starts = jnp.arange(NWIN + 1, dtype=jnp.int32) * WS
    bounds = jnp.searchsorted(simp_edge_batch, starts, side="left")
    bounds = bounds.astype(jnp.int32)
    bounds = jnp.concatenate(
        [bounds, jnp.zeros((BOUNDS_PAD - NWIN - 1,), jnp.int32)])
    out_e0, out_e1, out_attr, out_inv = _build(E)(
        edge_index[0], edge_index[1], edge_attr, simp_edge_batch, bounds)
    return jnp.stack([out_e0, out_e1]), out_attr, out_inv
